# Initial kernel scaffold; baseline (speedup 1.0000x reference)
import jax
import jax.numpy as jnp
from jax.experimental import pallas as pl


def kernel(pred_depth, gt_depth, grid, grid_shift):
    raise NotImplementedError("write your pallas kernel here")


# speedup vs baseline: 30.0631x; 30.0631x over previous
import functools

import jax
import jax.numpy as jnp
from jax import lax
from jax.experimental import pallas as pl
from jax.experimental.pallas import tpu as pltpu
from jax.experimental.pallas import tpu_sc as plsc

H, W = 1080, 1920
C = 3
WC = W * C
L = 16
NGRP = WC // L
R = 10
HALO = R + 20
NCHUNK = H // R
NC, NS = 2, 16
NW = NC * NS
CHUNKS_PER_TILE = (NCHUNK + NW - 1) // NW
TOL = 0.05


def _sc_body(gs_hbm, gt_hbm, pr_hbm, out_hbm, gtb, prb, gxb, gyb, accb):
    cid = lax.axis_index("c")
    sid = lax.axis_index("s")
    wid = sid * NC + cid

    zero = jnp.zeros((L,), jnp.float32)
    accb[pl.ds(0, L)] = zero
    accb[pl.ds(L, L)] = zero
    accb[pl.ds(2 * L, L)] = zero

    lane = lax.iota(jnp.int32, L)

    def do_chunk(chunk):
        base = chunk * R
        start = jnp.clip(base - 10, 0, H - HALO)
        pltpu.sync_copy(gt_hbm.at[pl.ds(start * W, HALO * W)], gtb)
        pltpu.sync_copy(pr_hbm.at[pl.ds(start * W, HALO * W)], prb)

        def row_body(r, carry):
            row = base + r
            pltpu.sync_copy(gs_hbm.at[pl.ds(row * WC, WC)], gxb)
            pltpu.sync_copy(gs_hbm.at[pl.ds((H + row) * WC, WC)], gyb)
            rowoff = (row - start) * W

            def grp(i, acc):
                a0, a1, a2 = acc
                col = i * L
                gxv = gxb[pl.ds(col, L)]
                gyv = gyb[pl.ds(col, L)]
                tidx = (gyv - start) * W + gxv
                tg = plsc.load_gather(gtb, [tidx])
                tp = plsc.load_gather(prb, [tidx])
                sidx = rowoff + lax.div(col + lane, 3)
                sg = plsc.load_gather(gtb, [sidx])
                sp = plsc.load_gather(prb, [sidx])

                rel = sg / (tg + 1e-8)
                pos = rel >= (1.0 + TOL)
                neg = rel <= 1.0 / (1.0 + TOL)
                sgn = jnp.where(pos, 1.0, 0.0) - jnp.where(neg, 1.0, 0.0)
                nz = sgn != 0.0
                diff = sp - tp
                t = -sgn * diff
                e = jnp.exp(-jnp.abs(diff))
                z = e / (2.0 + e)
                z2 = z * z
                p = z * (2.0 + z2 * (2.0 / 3.0 + z2 * (2.0 / 5.0 + z2 * (
                    2.0 / 7.0 + z2 * (2.0 / 9.0 + z2 * (2.0 / 11.0))))))
                soft = jnp.maximum(t, 0.0) + p
                a0 = a0 + jnp.where(nz, 1.0, 0.0)
                a1 = a1 + jnp.where(nz, soft, 0.0)
                a2 = a2 + jnp.where(nz, 0.0, diff * diff)
                return a0, a1, a2

            return lax.fori_loop(0, NGRP, grp, carry)

        a0, a1, a2 = lax.fori_loop(0, R, row_body, (zero, zero, zero))
        accb[pl.ds(0, L)] = accb[pl.ds(0, L)] + a0
        accb[pl.ds(L, L)] = accb[pl.ds(L, L)] + a1
        accb[pl.ds(2 * L, L)] = accb[pl.ds(2 * L, L)] + a2

    def chunk_body(ci, _):
        chunk = wid + ci * NW

        @pl.when(chunk < NCHUNK)
        def _():
            do_chunk(chunk)

        return 0

    lax.fori_loop(0, CHUNKS_PER_TILE, chunk_body, 0)
    pltpu.sync_copy(accb, out_hbm.at[pl.ds(wid * 3 * L, 3 * L)])


@functools.partial(
    pl.kernel,
    out_type=jax.ShapeDtypeStruct((NW * 3 * L,), jnp.float32),
    mesh=plsc.VectorSubcoreMesh(core_axis_name="c", subcore_axis_name="s"),
    compiler_params=pltpu.CompilerParams(needs_layout_passes=False),
    scratch_types=[
        pltpu.VMEM((HALO * W,), jnp.float32),
        pltpu.VMEM((HALO * W,), jnp.float32),
        pltpu.VMEM((WC,), jnp.int32),
        pltpu.VMEM((WC,), jnp.int32),
        pltpu.VMEM((3 * L,), jnp.float32),
    ],
)
def _depth_loss_partials(gs_hbm, gt_hbm, pr_hbm, out_hbm, gtb, prb, gxb, gyb,
                         accb):
    _sc_body(gs_hbm, gt_hbm, pr_hbm, out_hbm, gtb, prb, gxb, gyb, accb)


def kernel(pred_depth, gt_depth, grid, grid_shift):
    gs = grid_shift.reshape(2 * H * WC)
    parts = _depth_loss_partials(gs, gt_depth.reshape(H * W),
                                 pred_depth.reshape(H * W))
    parts = parts.reshape(NW, 3, L)
    n_nz = jnp.sum(parts[:, 0])
    s_soft = jnp.sum(parts[:, 1])
    s_sq = jnp.sum(parts[:, 2])
    total = jnp.float32(H * WC)
    depth_loss = s_soft / jnp.maximum(n_nz, 1.0)
    depth_loss_sim = s_sq / jnp.maximum(total - n_nz, 1.0)
    return depth_loss + depth_loss_sim
